# baseline (speedup 1.0000x reference)
import jax
import jax.numpy as jnp
from jax import lax
from jax.experimental import pallas as pl
from jax.experimental.pallas import tpu as pltpu
from jax.experimental.pallas import tpu_sc as plsc

NG = 100000
NSUB = 10000
NE = 160000
DM = 128
DF = 128
DMSG = 16
NBATCH = 2048
DZ = DM + DF
DH = 128

NC, NSC, LANES = 2, 16, 16
NW = NC * NSC
DHALF = DZ // NC

_f32 = jnp.float32
_i32 = jnp.int32


def _sc_mesh():
    return plsc.VectorSubcoreMesh(core_axis_name="c", subcore_axis_name="s",
                                  num_cores=NC, num_subcores=NSC)


_P0_K = 80
_P0_NBLK = NSUB // _P0_K


def _p0_body(nid_hbm, mem_hbm, x_hbm, lut_hbm, m_hbm, xg_hbm, lu_hbm,
             idx_v, mbuf, xbuf, lubuf, lut_tbl, sem):
    c = lax.axis_index("c")
    s = lax.axis_index("s")
    wid = s * NC + c
    pltpu.sync_copy(lut_hbm, lut_tbl)
    nblk = jnp.where(wid < _P0_NBLK - (_P0_NBLK // NW) * NW,
                     _P0_NBLK // NW + 1, _P0_NBLK // NW)

    def blk(i, carry):
        r0 = (wid + NW * i) * _P0_K
        pltpu.sync_copy(nid_hbm.at[pl.ds(r0, _P0_K)], idx_v)
        pltpu.async_copy(mem_hbm.at[idx_v], mbuf, sem).wait()
        pltpu.sync_copy(mbuf, m_hbm.at[pl.ds(r0, _P0_K), :])
        pltpu.async_copy(x_hbm.at[idx_v], xbuf, sem).wait()
        pltpu.sync_copy(xbuf, xg_hbm.at[pl.ds(r0, _P0_K), :])
        for j in range(_P0_K // LANES):
            sl = pl.ds(j * LANES, LANES)
            lubuf[sl] = plsc.load_gather(lut_tbl, [idx_v[sl]])
        pltpu.sync_copy(lubuf, lu_hbm.at[pl.ds(r0, _P0_K)])
        return carry

    lax.fori_loop(0, nblk, blk, 0)


def _p0(n_id, mem_table, x, last_update_table):
    return pl.kernel(
        _p0_body,
        out_type=(jax.ShapeDtypeStruct((NSUB, DM), _f32),
                  jax.ShapeDtypeStruct((NSUB, DF), _f32),
                  jax.ShapeDtypeStruct((NSUB,), _f32)),
        mesh=_sc_mesh(),
        compiler_params=pltpu.CompilerParams(needs_layout_passes=False),
        scratch_types=[
            pltpu.VMEM((_P0_K,), _i32),
            pltpu.VMEM((_P0_K, DM), _f32),
            pltpu.VMEM((_P0_K, DF), _f32),
            pltpu.VMEM((_P0_K,), _f32),
            pltpu.VMEM((NG,), _f32),
            pltpu.SemaphoreType.DMA,
        ],
    )(n_id, mem_table, x, last_update_table)


_P1T_BN = 2000


def _p1t_body(m_ref, xg_ref, lu_ref, a1, a2, wt_ref, s1, s2, zw_ref, zself_ref):
    mm = m_ref[...]
    xx = xg_ref[...]
    zw_ref[...] = mm @ a1[...] + xx @ a2[...] - lu_ref[...] * wt_ref[...]
    zself_ref[...] = mm @ s1[...] + xx @ s2[...]


def _p1t(m, xg, lu2, W_msg, W_self):
    nb = NSUB // _P1T_BN
    return pl.pallas_call(
        _p1t_body,
        grid=(NC, nb),
        in_specs=[
            pl.BlockSpec((_P1T_BN, DM), lambda c, i: (i, 0)),
            pl.BlockSpec((_P1T_BN, DF), lambda c, i: (i, 0)),
            pl.BlockSpec((_P1T_BN, 1), lambda c, i: (i, 0)),
            pl.BlockSpec((DM, DHALF), lambda c, i: (0, c)),
            pl.BlockSpec((DF, DHALF), lambda c, i: (0, c)),
            pl.BlockSpec((1, DHALF), lambda c, i: (0, c)),
            pl.BlockSpec((DM, DHALF), lambda c, i: (0, c)),
            pl.BlockSpec((DF, DHALF), lambda c, i: (0, c)),
        ],
        out_specs=(
            pl.BlockSpec((_P1T_BN, DHALF), lambda c, i: (c * (NSUB // _P1T_BN) + i, 0)),
            pl.BlockSpec((_P1T_BN, DHALF), lambda c, i: (i, c)),
        ),
        out_shape=(jax.ShapeDtypeStruct((NC * NSUB, DHALF), _f32),
                   jax.ShapeDtypeStruct((NSUB, DZ), _f32)),
    )(m, xg, lu2, W_msg[:DM], W_msg[DM:DZ], W_msg[DZ + DMSG].reshape(1, DZ),
      W_self[:DM], W_self[DM:])


_P1E_BE = 2000


def _p1e_body(msg_ref, c_ref, bm_ref, eb_ref):
    eb_ref[...] = msg_ref[...] @ c_ref[...] + bm_ref[...]


def _p1e(msg, W_msg, b_msg):
    ne = NE // _P1E_BE
    return pl.pallas_call(
        _p1e_body,
        grid=(NC, ne),
        in_specs=[
            pl.BlockSpec((_P1E_BE, DMSG), lambda c, i: (i, 0)),
            pl.BlockSpec((DMSG, DHALF), lambda c, i: (0, c)),
            pl.BlockSpec((1, DHALF), lambda c, i: (0, c)),
        ],
        out_specs=pl.BlockSpec((_P1E_BE, DHALF),
                               lambda c, i: (c * (NE // _P1E_BE) + i, 0)),
        out_shape=jax.ShapeDtypeStruct((NC * NE, DHALF), _f32),
    )(msg, W_msg[DZ:DZ + DMSG], b_msg.reshape(1, DZ))


_P2_K = 128
_P2_NBLK = NE // _P2_K
_P2_PER = _P2_NBLK // NSC
_P2_R0 = 624
_P2_CHUNKS = 5


def _p2_body(zw_hbm, eb_hbm, srce_hbm, dste_hbm, t_hbm, wt_hbm, agg_hbm,
             sidx, didx, buf, tbuf, wtbuf, aggsh, sem):
    c = lax.axis_index("c")
    s = lax.axis_index("s")
    pltpu.sync_copy(wt_hbm.at[pl.ds(c * DHALF, DHALF)], wtbuf)
    wvs = [wtbuf[pl.ds(j * LANES, LANES)] for j in range(DHALF // LANES)]

    def zrow(r, carry):
        for j in range(DHALF // LANES):
            buf[r, pl.ds(j * LANES, LANES)] = jnp.zeros((LANES,), _f32)
        return carry

    lax.fori_loop(0, _P2_K, zrow, 0)
    r0 = s * _P2_R0
    for k in range(_P2_CHUNKS):
        pltpu.sync_copy(buf, aggsh.at[pl.ds(r0 + k * _P2_K, _P2_K), :])
    plsc.subcore_barrier()

    nblk = jnp.where(s < _P2_NBLK - _P2_PER * NSC, _P2_PER + 1, _P2_PER)
    zoff = c * NSUB

    def blk(i, carry):
        e0 = (s + NSC * i) * _P2_K
        pltpu.sync_copy(srce_hbm.at[pl.ds(e0, _P2_K)], sidx)
        for j in range(_P2_K // LANES):
            sl = pl.ds(j * LANES, LANES)
            sidx[sl] = sidx[sl] + zoff
        pltpu.sync_copy(dste_hbm.at[pl.ds(e0, _P2_K)], didx)
        pltpu.sync_copy(eb_hbm.at[pl.ds(c * NE + e0, _P2_K), :], buf)
        pltpu.sync_copy(t_hbm.at[pl.ds(e0, _P2_K)], tbuf.at[pl.ds(0, _P2_K)])
        pltpu.async_copy(zw_hbm.at[sidx], buf, sem, add=True).wait()

        def rrow(r, carry2):
            tv = tbuf[pl.ds(r, LANES)][0]
            for j in range(DHALF // LANES):
                sl2 = pl.ds(j * LANES, LANES)
                buf[r, sl2] = jnp.maximum(buf[r, sl2] + tv * wvs[j], 0.0)
            return carry2

        lax.fori_loop(0, _P2_K, rrow, 0)
        pltpu.sync_copy(buf, aggsh.at[didx], add=True)
        return carry

    lax.fori_loop(0, nblk, blk, 0)
    plsc.subcore_barrier()

    for k in range(_P2_CHUNKS):
        pltpu.sync_copy(aggsh.at[pl.ds(r0 + k * _P2_K, _P2_K), :], buf)
        pltpu.sync_copy(buf, agg_hbm.at[pl.ds(c * NSUB + r0 + k * _P2_K, _P2_K), :])


def _p2(zw_st, eb_st, src_e, dst_e, t, wt):
    return pl.kernel(
        _p2_body,
        out_type=jax.ShapeDtypeStruct((NC * NSUB, DHALF), _f32),
        mesh=_sc_mesh(),
        compiler_params=pltpu.CompilerParams(needs_layout_passes=False),
        scratch_types=[
            pltpu.VMEM((_P2_K,), _i32),
            pltpu.VMEM((_P2_K,), _i32),
            pltpu.VMEM((_P2_K, DHALF), _f32),
            pltpu.VMEM((_P2_K + LANES,), _f32),
            pltpu.VMEM((DHALF,), _f32),
            pltpu.VMEM_SHARED((NSUB, DHALF), _f32),
            pltpu.SemaphoreType.DMA,
        ],
    )(zw_st, eb_st, src_e, dst_e, t, wt)


_P3_BN = 2000


def _p3_body(zself_ref, agg_ref, znew_ref):
    znew_ref[...] = jnp.maximum(zself_ref[...] + agg_ref[...], 0.0)


def _p3(zself, agg_st):
    nb = NSUB // _P3_BN
    return pl.pallas_call(
        _p3_body,
        grid=(NC, nb),
        in_specs=[
            pl.BlockSpec((_P3_BN, DHALF), lambda c, i: (i, c)),
            pl.BlockSpec((_P3_BN, DHALF), lambda c, i: (c * (NSUB // _P3_BN) + i, 0)),
        ],
        out_specs=pl.BlockSpec((_P3_BN, DHALF), lambda c, i: (i, c)),
        out_shape=jax.ShapeDtypeStruct((NSUB, DZ), _f32),
    )(zself, agg_st)


_P4_PB = NBATCH // NW
_P4_H = 32


def _p4_body(src_hbm, dst_hbm, nd_hbm, idmap_hbm, znew_hbm, m_hbm,
             zs_hbm, zd_hbm, znd_hbm, ms_hbm, md_hbm,
             idmap_tbl, iidx, gbuf, zrows, mrows, sem):
    c = lax.axis_index("c")
    s = lax.axis_index("s")
    wid = s * NC + c
    pltpu.sync_copy(idmap_hbm, idmap_tbl)

    def do(idx_hbm, zout, mout):
        for h in range(_P4_PB // _P4_H):
            b0 = wid * _P4_PB + h * _P4_H
            pltpu.sync_copy(idx_hbm.at[pl.ds(b0, _P4_H)], iidx)
            for j in range(_P4_H // LANES):
                sl = pl.ds(j * LANES, LANES)
                gbuf[sl] = plsc.load_gather(idmap_tbl, [iidx[sl]])
            pltpu.async_copy(znew_hbm.at[gbuf], zrows, sem).wait()
            pltpu.sync_copy(zrows, zout.at[pl.ds(b0, _P4_H), :])
            if mout is not None:
                pltpu.async_copy(m_hbm.at[gbuf], mrows, sem).wait()
                pltpu.sync_copy(mrows, mout.at[pl.ds(b0, _P4_H), :])

    do(src_hbm, zs_hbm, ms_hbm)
    do(dst_hbm, zd_hbm, md_hbm)
    do(nd_hbm, znd_hbm, None)


def _p4(src, dst, neg_dst, id_mapper, znew, m):
    return pl.kernel(
        _p4_body,
        out_type=(jax.ShapeDtypeStruct((NBATCH, DZ), _f32),
                  jax.ShapeDtypeStruct((NBATCH, DZ), _f32),
                  jax.ShapeDtypeStruct((NBATCH, DZ), _f32),
                  jax.ShapeDtypeStruct((NBATCH, DM), _f32),
                  jax.ShapeDtypeStruct((NBATCH, DM), _f32)),
        mesh=_sc_mesh(),
        compiler_params=pltpu.CompilerParams(needs_layout_passes=False),
        scratch_types=[
            pltpu.VMEM((NG,), _i32),
            pltpu.VMEM((_P4_H,), _i32),
            pltpu.VMEM((_P4_H,), _i32),
            pltpu.VMEM((_P4_H, DZ), _f32),
            pltpu.VMEM((_P4_H, DM), _f32),
            pltpu.SemaphoreType.DMA,
        ],
    )(src, dst, neg_dst, id_mapper, znew, m)


def _p5_body(zs_ref, zd_ref, znd_ref, w1t, w1b, b1_ref, w2_ref, b2_ref,
             pos_ref, neg_ref):
    zs = zs_ref[...]
    b2 = b2_ref[0, 0]
    hp = jnp.maximum(zs @ w1t[...] + zd_ref[...] @ w1b[...] + b1_ref[...], 0.0)
    pos_ref[...] = hp @ w2_ref[...] + b2
    hn = jnp.maximum(zs @ w1t[...] + znd_ref[...] @ w1b[...] + b1_ref[...], 0.0)
    neg_ref[...] = hn @ w2_ref[...] + b2


def _p5(zs, zd, znd, W1, b1, W2, b2):
    return pl.pallas_call(
        _p5_body,
        out_shape=(jax.ShapeDtypeStruct((NBATCH, 1), _f32),
                   jax.ShapeDtypeStruct((NBATCH, 1), _f32)),
    )(zs, zd, znd, W1[:DZ], W1[DZ:], b1.reshape(1, DH), W2,
      b2.reshape(1, 1))


def kernel(src, dst, neg_dst, x, n_id, msg, t, edge_index, id_mapper,
           mem_table, last_update_table, W_msg, b_msg, W_self, W1, b1, W2, b2):
    src_e = edge_index[0]
    dst_e = edge_index[1]
    m, xg, lu = _p0(n_id, mem_table, x, last_update_table)
    zw_st, zself = _p1t(m, xg, lu.reshape(NSUB, 1), W_msg, W_self)
    eb_st = _p1e(msg, W_msg, b_msg)
    agg_st = _p2(zw_st, eb_st, src_e, dst_e, t, W_msg[DZ + DMSG])
    znew = _p3(zself, agg_st)
    zs, zd, znd, ms, md = _p4(src, dst, neg_dst, id_mapper, znew, m)
    pos, neg = _p5(zs, zd, znd, W1, b1, W2, b2)
    return pos, neg, ms, md


# speedup vs baseline: 1.2800x; 1.2800x over previous
import jax
import jax.numpy as jnp
from jax import lax
from jax.experimental import pallas as pl
from jax.experimental.pallas import tpu as pltpu
from jax.experimental.pallas import tpu_sc as plsc

NG = 100000
NSUB = 10000
NE = 160000
DM = 128
DF = 128
DMSG = 16
NBATCH = 2048
DZ = DM + DF
DH = 128

NC, NSC, LANES = 2, 16, 16
NW = NC * NSC
DHALF = DZ // NC

_f32 = jnp.float32
_i32 = jnp.int32


def _sc_mesh():
    return plsc.VectorSubcoreMesh(core_axis_name="c", subcore_axis_name="s",
                                  num_cores=NC, num_subcores=NSC)


_P0_K = 80
_P0_NBLK = NSUB // _P0_K


def _p0_body(nid_hbm, mem_hbm, x_hbm, lut_hbm, m_hbm, xg_hbm, lu_hbm,
             idx_v, mbuf, xbuf, lubuf, lut_tbl, sem):
    c = lax.axis_index("c")
    s = lax.axis_index("s")
    wid = s * NC + c
    pltpu.sync_copy(lut_hbm, lut_tbl)
    nblk = jnp.where(wid < _P0_NBLK - (_P0_NBLK // NW) * NW,
                     _P0_NBLK // NW + 1, _P0_NBLK // NW)

    def blk(i, carry):
        r0 = (wid + NW * i) * _P0_K
        pltpu.sync_copy(nid_hbm.at[pl.ds(r0, _P0_K)], idx_v)
        pltpu.async_copy(mem_hbm.at[idx_v], mbuf, sem).wait()
        pltpu.sync_copy(mbuf, m_hbm.at[pl.ds(r0, _P0_K), :])
        pltpu.async_copy(x_hbm.at[idx_v], xbuf, sem).wait()
        pltpu.sync_copy(xbuf, xg_hbm.at[pl.ds(r0, _P0_K), :])
        for j in range(_P0_K // LANES):
            sl = pl.ds(j * LANES, LANES)
            lubuf[sl] = plsc.load_gather(lut_tbl, [idx_v[sl]])
        pltpu.sync_copy(lubuf, lu_hbm.at[pl.ds(r0, _P0_K)])
        return carry

    lax.fori_loop(0, nblk, blk, 0)


def _p0(n_id, mem_table, x, last_update_table):
    return pl.kernel(
        _p0_body,
        out_type=(jax.ShapeDtypeStruct((NSUB, DM), _f32),
                  jax.ShapeDtypeStruct((NSUB, DF), _f32),
                  jax.ShapeDtypeStruct((NSUB,), _f32)),
        mesh=_sc_mesh(),
        compiler_params=pltpu.CompilerParams(needs_layout_passes=False),
        scratch_types=[
            pltpu.VMEM((_P0_K,), _i32),
            pltpu.VMEM((_P0_K, DM), _f32),
            pltpu.VMEM((_P0_K, DF), _f32),
            pltpu.VMEM((_P0_K,), _f32),
            pltpu.VMEM((NG,), _f32),
            pltpu.SemaphoreType.DMA,
        ],
    )(n_id, mem_table, x, last_update_table)


_P1T_BN = 2000


def _p1t_body(m_ref, xg_ref, lu_ref, a1, a2, wt_ref, s1, s2, zw_ref, zself_ref):
    mm = m_ref[...]
    xx = xg_ref[...]
    zw_ref[...] = mm @ a1[...] + xx @ a2[...] - lu_ref[...] * wt_ref[...]
    zself_ref[...] = mm @ s1[...] + xx @ s2[...]


def _p1t(m, xg, lu2, W_msg, W_self):
    nb = NSUB // _P1T_BN
    return pl.pallas_call(
        _p1t_body,
        grid=(NC, nb),
        in_specs=[
            pl.BlockSpec((_P1T_BN, DM), lambda c, i: (i, 0)),
            pl.BlockSpec((_P1T_BN, DF), lambda c, i: (i, 0)),
            pl.BlockSpec((_P1T_BN, 1), lambda c, i: (i, 0)),
            pl.BlockSpec((DM, DHALF), lambda c, i: (0, c)),
            pl.BlockSpec((DF, DHALF), lambda c, i: (0, c)),
            pl.BlockSpec((1, DHALF), lambda c, i: (0, c)),
            pl.BlockSpec((DM, DHALF), lambda c, i: (0, c)),
            pl.BlockSpec((DF, DHALF), lambda c, i: (0, c)),
        ],
        out_specs=(
            pl.BlockSpec((_P1T_BN, DHALF), lambda c, i: (c * (NSUB // _P1T_BN) + i, 0)),
            pl.BlockSpec((_P1T_BN, DHALF), lambda c, i: (i, c)),
        ),
        out_shape=(jax.ShapeDtypeStruct((NC * NSUB, DHALF), _f32),
                   jax.ShapeDtypeStruct((NSUB, DZ), _f32)),
    )(m, xg, lu2, W_msg[:DM], W_msg[DM:DZ], W_msg[DZ + DMSG].reshape(1, DZ),
      W_self[:DM], W_self[DM:])


_P1E_BE = 2000


def _p1e_body(msg_ref, c_ref, bm_ref, eb_ref):
    eb_ref[...] = msg_ref[...] @ c_ref[...] + bm_ref[...]


def _p1e(msg, W_msg, b_msg):
    ne = NE // _P1E_BE
    return pl.pallas_call(
        _p1e_body,
        grid=(NC, ne),
        in_specs=[
            pl.BlockSpec((_P1E_BE, DMSG), lambda c, i: (i, 0)),
            pl.BlockSpec((DMSG, DHALF), lambda c, i: (0, c)),
            pl.BlockSpec((1, DHALF), lambda c, i: (0, c)),
        ],
        out_specs=pl.BlockSpec((_P1E_BE, DHALF),
                               lambda c, i: (c * (NE // _P1E_BE) + i, 0)),
        out_shape=jax.ShapeDtypeStruct((NC * NE, DHALF), _f32),
    )(msg, W_msg[DZ:DZ + DMSG], b_msg.reshape(1, DZ))


_P2_K = 128
_P2_NBLK = NE // _P2_K
_P2_WAVE = 3
_P2_NWAVES = 26
_P2_EXTRA0 = _P2_NWAVES * _P2_WAVE * NSC
_P2_R0 = 624
_P2_CHUNKS = 5


def _p2_body(zw_hbm, eb_hbm, srce_hbm, dste_hbm, t_hbm, wt_hbm, agg_hbm, *scr):
    W = _P2_WAVE
    sidx = scr[0:W]
    didx = scr[W:2 * W]
    buf = scr[2 * W:3 * W]
    tbuf = scr[3 * W:4 * W]
    wtbuf = scr[4 * W]
    aggsh = scr[4 * W + 1]
    lsem = scr[4 * W + 2:5 * W + 2]
    gsem = scr[5 * W + 2:6 * W + 2]
    c = lax.axis_index("c")
    s = lax.axis_index("s")
    pltpu.sync_copy(wt_hbm.at[pl.ds(c * DHALF, DHALF)], wtbuf)
    wvs = [wtbuf[pl.ds(j * LANES, LANES)] for j in range(DHALF // LANES)]

    def zrow(r, carry):
        for j in range(DHALF // LANES):
            buf[0][r, pl.ds(j * LANES, LANES)] = jnp.zeros((LANES,), _f32)
        return carry

    lax.fori_loop(0, _P2_K, zrow, 0)
    r0 = s * _P2_R0
    for k in range(_P2_CHUNKS):
        pltpu.sync_copy(buf[0], aggsh.at[pl.ds(r0 + k * _P2_K, _P2_K), :])
    plsc.subcore_barrier()

    zoff = c * NSUB

    def fire_loads(k, e0):
        return [
            pltpu.async_copy(srce_hbm.at[pl.ds(e0, _P2_K)], sidx[k], lsem[k]),
            pltpu.async_copy(dste_hbm.at[pl.ds(e0, _P2_K)], didx[k], lsem[k]),
            pltpu.async_copy(eb_hbm.at[pl.ds(c * NE + e0, _P2_K), :], buf[k],
                             lsem[k]),
            pltpu.async_copy(t_hbm.at[pl.ds(e0, _P2_K)],
                             tbuf[k].at[pl.ds(0, _P2_K)], lsem[k]),
        ]

    def compute_block(k):
        def rrow(r, carry2):
            tv = tbuf[k][pl.ds(r, LANES)][0]
            for j in range(DHALF // LANES):
                sl2 = pl.ds(j * LANES, LANES)
                buf[k][r, sl2] = jnp.maximum(buf[k][r, sl2] + tv * wvs[j], 0.0)
            return carry2

        lax.fori_loop(0, _P2_K, rrow, 0)

    def wave(g, carry):
        descs = []
        for k in range(W):
            e0 = (s + NSC * (W * g + k)) * _P2_K
            descs.append(fire_loads(k, e0))
        gds = []
        for k in range(W):
            for d in descs[k]:
                d.wait()
            for j in range(_P2_K // LANES):
                sl = pl.ds(j * LANES, LANES)
                sidx[k][sl] = sidx[k][sl] + zoff
            gds.append(pltpu.async_copy(zw_hbm.at[sidx[k]], buf[k], gsem[k],
                                        add=True))
        for k in range(W):
            gds[k].wait()
            compute_block(k)
            pltpu.sync_copy(buf[k], aggsh.at[didx[k]], add=True)
        return carry

    lax.fori_loop(0, _P2_NWAVES, wave, 0)

    def tail(i, carry):
        e0 = (_P2_EXTRA0 + s) * _P2_K
        for d in fire_loads(0, e0):
            d.wait()
        for j in range(_P2_K // LANES):
            sl = pl.ds(j * LANES, LANES)
            sidx[0][sl] = sidx[0][sl] + zoff
        pltpu.async_copy(zw_hbm.at[sidx[0]], buf[0], gsem[0], add=True).wait()
        compute_block(0)
        pltpu.sync_copy(buf[0], aggsh.at[didx[0]], add=True)
        return carry

    lax.fori_loop(0, jnp.where(s < _P2_NBLK - _P2_EXTRA0, 1, 0), tail, 0)
    plsc.subcore_barrier()

    for k in range(_P2_CHUNKS):
        pltpu.sync_copy(aggsh.at[pl.ds(r0 + k * _P2_K, _P2_K), :], buf[0])
        pltpu.sync_copy(buf[0],
                        agg_hbm.at[pl.ds(c * NSUB + r0 + k * _P2_K, _P2_K), :])


def _p2(zw_st, eb_st, src_e, dst_e, t, wt):
    W = _P2_WAVE
    scratch = (
        [pltpu.VMEM((_P2_K,), _i32) for _ in range(W)]
        + [pltpu.VMEM((_P2_K,), _i32) for _ in range(W)]
        + [pltpu.VMEM((_P2_K, DHALF), _f32) for _ in range(W)]
        + [pltpu.VMEM((_P2_K + LANES,), _f32) for _ in range(W)]
        + [pltpu.VMEM((DHALF,), _f32)]
        + [pltpu.VMEM_SHARED((NSUB, DHALF), _f32)]
        + [pltpu.SemaphoreType.DMA for _ in range(2 * W)]
    )
    return pl.kernel(
        _p2_body,
        out_type=jax.ShapeDtypeStruct((NC * NSUB, DHALF), _f32),
        mesh=_sc_mesh(),
        compiler_params=pltpu.CompilerParams(needs_layout_passes=False),
        scratch_types=scratch,
    )(zw_st, eb_st, src_e, dst_e, t, wt)


_P3_BN = 2000


def _p3_body(zself_ref, agg_ref, znew_ref):
    znew_ref[...] = jnp.maximum(zself_ref[...] + agg_ref[...], 0.0)


def _p3(zself, agg_st):
    nb = NSUB // _P3_BN
    return pl.pallas_call(
        _p3_body,
        grid=(NC, nb),
        in_specs=[
            pl.BlockSpec((_P3_BN, DHALF), lambda c, i: (i, c)),
            pl.BlockSpec((_P3_BN, DHALF), lambda c, i: (c * (NSUB // _P3_BN) + i, 0)),
        ],
        out_specs=pl.BlockSpec((_P3_BN, DHALF), lambda c, i: (i, c)),
        out_shape=jax.ShapeDtypeStruct((NSUB, DZ), _f32),
    )(zself, agg_st)


_P4_PB = NBATCH // NW
_P4_H = 32


def _p4_body(src_hbm, dst_hbm, nd_hbm, idmap_hbm, znew_hbm, m_hbm,
             zs_hbm, zd_hbm, znd_hbm, ms_hbm, md_hbm,
             idmap_tbl, iidx, gbuf, zrows, mrows, sem):
    c = lax.axis_index("c")
    s = lax.axis_index("s")
    wid = s * NC + c
    pltpu.sync_copy(idmap_hbm, idmap_tbl)

    def do(idx_hbm, zout, mout):
        for h in range(_P4_PB // _P4_H):
            b0 = wid * _P4_PB + h * _P4_H
            pltpu.sync_copy(idx_hbm.at[pl.ds(b0, _P4_H)], iidx)
            for j in range(_P4_H // LANES):
                sl = pl.ds(j * LANES, LANES)
                gbuf[sl] = plsc.load_gather(idmap_tbl, [iidx[sl]])
            pltpu.async_copy(znew_hbm.at[gbuf], zrows, sem).wait()
            pltpu.sync_copy(zrows, zout.at[pl.ds(b0, _P4_H), :])
            if mout is not None:
                pltpu.async_copy(m_hbm.at[gbuf], mrows, sem).wait()
                pltpu.sync_copy(mrows, mout.at[pl.ds(b0, _P4_H), :])

    do(src_hbm, zs_hbm, ms_hbm)
    do(dst_hbm, zd_hbm, md_hbm)
    do(nd_hbm, znd_hbm, None)


def _p4(src, dst, neg_dst, id_mapper, znew, m):
    return pl.kernel(
        _p4_body,
        out_type=(jax.ShapeDtypeStruct((NBATCH, DZ), _f32),
                  jax.ShapeDtypeStruct((NBATCH, DZ), _f32),
                  jax.ShapeDtypeStruct((NBATCH, DZ), _f32),
                  jax.ShapeDtypeStruct((NBATCH, DM), _f32),
                  jax.ShapeDtypeStruct((NBATCH, DM), _f32)),
        mesh=_sc_mesh(),
        compiler_params=pltpu.CompilerParams(needs_layout_passes=False),
        scratch_types=[
            pltpu.VMEM((NG,), _i32),
            pltpu.VMEM((_P4_H,), _i32),
            pltpu.VMEM((_P4_H,), _i32),
            pltpu.VMEM((_P4_H, DZ), _f32),
            pltpu.VMEM((_P4_H, DM), _f32),
            pltpu.SemaphoreType.DMA,
        ],
    )(src, dst, neg_dst, id_mapper, znew, m)


def _p5_body(zs_ref, zd_ref, znd_ref, w1t, w1b, b1_ref, w2_ref, b2_ref,
             pos_ref, neg_ref):
    zs = zs_ref[...]
    b2 = b2_ref[0, 0]
    hp = jnp.maximum(zs @ w1t[...] + zd_ref[...] @ w1b[...] + b1_ref[...], 0.0)
    pos_ref[...] = hp @ w2_ref[...] + b2
    hn = jnp.maximum(zs @ w1t[...] + znd_ref[...] @ w1b[...] + b1_ref[...], 0.0)
    neg_ref[...] = hn @ w2_ref[...] + b2


def _p5(zs, zd, znd, W1, b1, W2, b2):
    return pl.pallas_call(
        _p5_body,
        out_shape=(jax.ShapeDtypeStruct((NBATCH, 1), _f32),
                   jax.ShapeDtypeStruct((NBATCH, 1), _f32)),
    )(zs, zd, znd, W1[:DZ], W1[DZ:], b1.reshape(1, DH), W2,
      b2.reshape(1, 1))


def kernel(src, dst, neg_dst, x, n_id, msg, t, edge_index, id_mapper,
           mem_table, last_update_table, W_msg, b_msg, W_self, W1, b1, W2, b2):
    src_e = edge_index[0]
    dst_e = edge_index[1]
    m, xg, lu = _p0(n_id, mem_table, x, last_update_table)
    zw_st, zself = _p1t(m, xg, lu.reshape(NSUB, 1), W_msg, W_self)
    eb_st = _p1e(msg, W_msg, b_msg)
    agg_st = _p2(zw_st, eb_st, src_e, dst_e, t, W_msg[DZ + DMSG])
    znew = _p3(zself, agg_st)
    zs, zd, znd, ms, md = _p4(src, dst, neg_dst, id_mapper, znew, m)
    pos, neg = _p5(zs, zd, znd, W1, b1, W2, b2)
    return pos, neg, ms, md


# speedup vs baseline: 1.4225x; 1.1114x over previous
import jax
import jax.numpy as jnp
from jax import lax
from jax.experimental import pallas as pl
from jax.experimental.pallas import tpu as pltpu
from jax.experimental.pallas import tpu_sc as plsc

NG = 100000
NSUB = 10000
NE = 160000
DM = 128
DF = 128
DMSG = 16
NBATCH = 2048
DZ = DM + DF
DH = 128

NC, NSC, LANES = 2, 16, 16
NW = NC * NSC
DHALF = DZ // NC

_f32 = jnp.float32
_i32 = jnp.int32


def _sc_mesh():
    return plsc.VectorSubcoreMesh(core_axis_name="c", subcore_axis_name="s",
                                  num_cores=NC, num_subcores=NSC)


_P0_K = 80
_P0_NBLK = NSUB // _P0_K


def _p0_body(nid_hbm, mem_hbm, x_hbm, lut_hbm, m_hbm, xg_hbm, lu_hbm,
             idx_v, mbuf, xbuf, lubuf, lut_tbl, sem):
    c = lax.axis_index("c")
    s = lax.axis_index("s")
    wid = s * NC + c
    pltpu.sync_copy(lut_hbm, lut_tbl)
    nblk = jnp.where(wid < _P0_NBLK - (_P0_NBLK // NW) * NW,
                     _P0_NBLK // NW + 1, _P0_NBLK // NW)

    def blk(i, carry):
        r0 = (wid + NW * i) * _P0_K
        pltpu.sync_copy(nid_hbm.at[pl.ds(r0, _P0_K)], idx_v)
        pltpu.async_copy(mem_hbm.at[idx_v], mbuf, sem).wait()
        pltpu.sync_copy(mbuf, m_hbm.at[pl.ds(r0, _P0_K), :])
        pltpu.async_copy(x_hbm.at[idx_v], xbuf, sem).wait()
        pltpu.sync_copy(xbuf, xg_hbm.at[pl.ds(r0, _P0_K), :])
        for j in range(_P0_K // LANES):
            sl = pl.ds(j * LANES, LANES)
            lubuf[sl] = plsc.load_gather(lut_tbl, [idx_v[sl]])
        pltpu.sync_copy(lubuf, lu_hbm.at[pl.ds(r0, _P0_K)])
        return carry

    lax.fori_loop(0, nblk, blk, 0)


def _p0(n_id, mem_table, x, last_update_table):
    return pl.kernel(
        _p0_body,
        out_type=(jax.ShapeDtypeStruct((NSUB, DM), _f32),
                  jax.ShapeDtypeStruct((NSUB, DF), _f32),
                  jax.ShapeDtypeStruct((NSUB,), _f32)),
        mesh=_sc_mesh(),
        compiler_params=pltpu.CompilerParams(needs_layout_passes=False),
        scratch_types=[
            pltpu.VMEM((_P0_K,), _i32),
            pltpu.VMEM((_P0_K, DM), _f32),
            pltpu.VMEM((_P0_K, DF), _f32),
            pltpu.VMEM((_P0_K,), _f32),
            pltpu.VMEM((NG,), _f32),
            pltpu.SemaphoreType.DMA,
        ],
    )(n_id, mem_table, x, last_update_table)


_P1T_BN = 2000


def _p1t_body(m_ref, xg_ref, lu_ref, a1, a2, wt_ref, s1, s2, zw_ref, zself_ref):
    mm = m_ref[...]
    xx = xg_ref[...]
    zw_ref[...] = mm @ a1[...] + xx @ a2[...] - lu_ref[...] * wt_ref[...]
    zself_ref[...] = mm @ s1[...] + xx @ s2[...]


def _p1t(m, xg, lu2, W_msg, W_self):
    nb = NSUB // _P1T_BN
    return pl.pallas_call(
        _p1t_body,
        grid=(NC, nb),
        in_specs=[
            pl.BlockSpec((_P1T_BN, DM), lambda c, i: (i, 0)),
            pl.BlockSpec((_P1T_BN, DF), lambda c, i: (i, 0)),
            pl.BlockSpec((_P1T_BN, 1), lambda c, i: (i, 0)),
            pl.BlockSpec((DM, DHALF), lambda c, i: (0, c)),
            pl.BlockSpec((DF, DHALF), lambda c, i: (0, c)),
            pl.BlockSpec((1, DHALF), lambda c, i: (0, c)),
            pl.BlockSpec((DM, DHALF), lambda c, i: (0, c)),
            pl.BlockSpec((DF, DHALF), lambda c, i: (0, c)),
        ],
        out_specs=(
            pl.BlockSpec((_P1T_BN, DHALF), lambda c, i: (c * (NSUB // _P1T_BN) + i, 0)),
            pl.BlockSpec((_P1T_BN, DHALF), lambda c, i: (i, c)),
        ),
        out_shape=(jax.ShapeDtypeStruct((NC * NSUB, DHALF), _f32),
                   jax.ShapeDtypeStruct((NSUB, DZ), _f32)),
    )(m, xg, lu2, W_msg[:DM], W_msg[DM:DZ], W_msg[DZ + DMSG].reshape(1, DZ),
      W_self[:DM], W_self[DM:])


_P1E_BE = 2000


def _p1e_body(msg_ref, c_ref, bm_ref, eb_ref):
    eb_ref[...] = msg_ref[...] @ c_ref[...] + bm_ref[...]


def _p1e(msg, W_msg, b_msg):
    ne = NE // _P1E_BE
    return pl.pallas_call(
        _p1e_body,
        grid=(ne,),
        in_specs=[
            pl.BlockSpec((_P1E_BE, DMSG), lambda i: (i, 0)),
            pl.BlockSpec((DMSG, DZ), lambda i: (0, 0)),
            pl.BlockSpec((1, DZ), lambda i: (0, 0)),
        ],
        out_specs=pl.BlockSpec((_P1E_BE, DZ), lambda i: (i, 0)),
        out_shape=jax.ShapeDtypeStruct((NE, DZ), _f32),
    )(msg, W_msg[DZ:DZ + DMSG], b_msg.reshape(1, DZ))


_P2_K = 128
_P2_NBLK = NE // _P2_K
_P2_WAVE = 3
_P2_NWAVES = 26
_P2_EXTRA0 = _P2_NWAVES * _P2_WAVE * NSC
_P2_R0 = 624
_P2_CHUNKS = 5


def _p2_body(zw_hbm, eb_hbm, srce_hbm, dste_hbm, t_hbm, wt_hbm, agg_hbm, *scr):
    W = _P2_WAVE
    sidx = scr[0:W]
    didx = scr[W:2 * W]
    buf = scr[2 * W:3 * W]
    tbuf = scr[3 * W:4 * W]
    wtbuf = scr[4 * W]
    aggsh = scr[4 * W + 1]
    lsem = scr[4 * W + 2:5 * W + 2]
    gsem = scr[5 * W + 2:6 * W + 2]
    c = lax.axis_index("c")
    s = lax.axis_index("s")
    pltpu.sync_copy(wt_hbm.at[pl.ds(c * DHALF, DHALF)], wtbuf)
    wvs = [wtbuf[pl.ds(j * LANES, LANES)] for j in range(DHALF // LANES)]

    def zrow(r, carry):
        for j in range(DHALF // LANES):
            buf[0][r, pl.ds(j * LANES, LANES)] = jnp.zeros((LANES,), _f32)
        return carry

    lax.fori_loop(0, _P2_K, zrow, 0)
    r0 = s * _P2_R0
    for k in range(_P2_CHUNKS):
        pltpu.sync_copy(buf[0], aggsh.at[pl.ds(r0 + k * _P2_K, _P2_K), :])
    plsc.subcore_barrier()

    zoff = c * NSUB

    def fire_loads(k, e0):
        return [
            pltpu.async_copy(srce_hbm.at[pl.ds(e0, _P2_K)], sidx[k], lsem[k]),
            pltpu.async_copy(dste_hbm.at[pl.ds(e0, _P2_K)], didx[k], lsem[k]),
            pltpu.async_copy(
                eb_hbm.at[pl.ds(e0, _P2_K), pl.ds(c * DHALF, DHALF)], buf[k],
                lsem[k]),
            pltpu.async_copy(t_hbm.at[pl.ds(e0, _P2_K)],
                             tbuf[k].at[pl.ds(0, _P2_K)], lsem[k]),
        ]

    def compute_block(k):
        def rrow(r, carry2):
            tv = tbuf[k][pl.ds(r, LANES)][0]
            for j in range(DHALF // LANES):
                sl2 = pl.ds(j * LANES, LANES)
                buf[k][r, sl2] = jnp.maximum(buf[k][r, sl2] + tv * wvs[j], 0.0)
            return carry2

        lax.fori_loop(0, _P2_K, rrow, 0)

    def wave(g, carry):
        descs = []
        for k in range(W):
            e0 = (s + NSC * (W * g + k)) * _P2_K
            descs.append(fire_loads(k, e0))
        gds = []
        for k in range(W):
            for d in descs[k]:
                d.wait()
            for j in range(_P2_K // LANES):
                sl = pl.ds(j * LANES, LANES)
                sidx[k][sl] = sidx[k][sl] + zoff
            gds.append(pltpu.async_copy(zw_hbm.at[sidx[k]], buf[k], gsem[k],
                                        add=True))
        for k in range(W):
            gds[k].wait()
            compute_block(k)
            pltpu.sync_copy(buf[k], aggsh.at[didx[k]], add=True)
        return carry

    lax.fori_loop(0, _P2_NWAVES, wave, 0)

    def tail(i, carry):
        e0 = (_P2_EXTRA0 + s) * _P2_K
        for d in fire_loads(0, e0):
            d.wait()
        for j in range(_P2_K // LANES):
            sl = pl.ds(j * LANES, LANES)
            sidx[0][sl] = sidx[0][sl] + zoff
        pltpu.async_copy(zw_hbm.at[sidx[0]], buf[0], gsem[0], add=True).wait()
        compute_block(0)
        pltpu.sync_copy(buf[0], aggsh.at[didx[0]], add=True)
        return carry

    lax.fori_loop(0, jnp.where(s < _P2_NBLK - _P2_EXTRA0, 1, 0), tail, 0)
    plsc.subcore_barrier()

    for k in range(_P2_CHUNKS):
        pltpu.sync_copy(aggsh.at[pl.ds(r0 + k * _P2_K, _P2_K), :], buf[0])
        pltpu.sync_copy(buf[0],
                        agg_hbm.at[pl.ds(c * NSUB + r0 + k * _P2_K, _P2_K), :])


def _p2(zw_st, eb_st, src_e, dst_e, t, wt):
    W = _P2_WAVE
    scratch = (
        [pltpu.VMEM((_P2_K,), _i32) for _ in range(W)]
        + [pltpu.VMEM((_P2_K,), _i32) for _ in range(W)]
        + [pltpu.VMEM((_P2_K, DHALF), _f32) for _ in range(W)]
        + [pltpu.VMEM((_P2_K + LANES,), _f32) for _ in range(W)]
        + [pltpu.VMEM((DHALF,), _f32)]
        + [pltpu.VMEM_SHARED((NSUB, DHALF), _f32)]
        + [pltpu.SemaphoreType.DMA for _ in range(2 * W)]
    )
    return pl.kernel(
        _p2_body,
        out_type=jax.ShapeDtypeStruct((NC * NSUB, DHALF), _f32),
        mesh=_sc_mesh(),
        compiler_params=pltpu.CompilerParams(needs_layout_passes=False),
        scratch_types=scratch,
    )(zw_st, eb_st, src_e, dst_e, t, wt)


_P3_BN = 2000


def _p3_body(zself_ref, agg_ref, znew_ref):
    znew_ref[...] = jnp.maximum(zself_ref[...] + agg_ref[...], 0.0)


def _p3(zself, agg_st):
    nb = NSUB // _P3_BN
    return pl.pallas_call(
        _p3_body,
        grid=(NC, nb),
        in_specs=[
            pl.BlockSpec((_P3_BN, DHALF), lambda c, i: (i, c)),
            pl.BlockSpec((_P3_BN, DHALF), lambda c, i: (c * (NSUB // _P3_BN) + i, 0)),
        ],
        out_specs=pl.BlockSpec((_P3_BN, DHALF), lambda c, i: (i, c)),
        out_shape=jax.ShapeDtypeStruct((NSUB, DZ), _f32),
    )(zself, agg_st)


_P4_PB = NBATCH // NW
_P4_H = 32


def _p4_body(src_hbm, dst_hbm, nd_hbm, idmap_hbm, znew_hbm, m_hbm,
             zs_hbm, zd_hbm, znd_hbm, ms_hbm, md_hbm,
             idmap_tbl, iidx, gbuf, zrows, mrows, sem):
    c = lax.axis_index("c")
    s = lax.axis_index("s")
    wid = s * NC + c
    pltpu.sync_copy(idmap_hbm, idmap_tbl)

    def do(idx_hbm, zout, mout):
        for h in range(_P4_PB // _P4_H):
            b0 = wid * _P4_PB + h * _P4_H
            pltpu.sync_copy(idx_hbm.at[pl.ds(b0, _P4_H)], iidx)
            for j in range(_P4_H // LANES):
                sl = pl.ds(j * LANES, LANES)
                gbuf[sl] = plsc.load_gather(idmap_tbl, [iidx[sl]])
            pltpu.async_copy(znew_hbm.at[gbuf], zrows, sem).wait()
            pltpu.sync_copy(zrows, zout.at[pl.ds(b0, _P4_H), :])
            if mout is not None:
                pltpu.async_copy(m_hbm.at[gbuf], mrows, sem).wait()
                pltpu.sync_copy(mrows, mout.at[pl.ds(b0, _P4_H), :])

    do(src_hbm, zs_hbm, ms_hbm)
    do(dst_hbm, zd_hbm, md_hbm)
    do(nd_hbm, znd_hbm, None)


def _p4(src, dst, neg_dst, id_mapper, znew, m):
    return pl.kernel(
        _p4_body,
        out_type=(jax.ShapeDtypeStruct((NBATCH, DZ), _f32),
                  jax.ShapeDtypeStruct((NBATCH, DZ), _f32),
                  jax.ShapeDtypeStruct((NBATCH, DZ), _f32),
                  jax.ShapeDtypeStruct((NBATCH, DM), _f32),
                  jax.ShapeDtypeStruct((NBATCH, DM), _f32)),
        mesh=_sc_mesh(),
        compiler_params=pltpu.CompilerParams(needs_layout_passes=False),
        scratch_types=[
            pltpu.VMEM((NG,), _i32),
            pltpu.VMEM((_P4_H,), _i32),
            pltpu.VMEM((_P4_H,), _i32),
            pltpu.VMEM((_P4_H, DZ), _f32),
            pltpu.VMEM((_P4_H, DM), _f32),
            pltpu.SemaphoreType.DMA,
        ],
    )(src, dst, neg_dst, id_mapper, znew, m)


def _p5_body(zs_ref, zd_ref, znd_ref, w1t, w1b, b1_ref, w2_ref, b2_ref,
             pos_ref, neg_ref):
    zs = zs_ref[...]
    b2 = b2_ref[0, 0]
    hp = jnp.maximum(zs @ w1t[...] + zd_ref[...] @ w1b[...] + b1_ref[...], 0.0)
    pos_ref[...] = hp @ w2_ref[...] + b2
    hn = jnp.maximum(zs @ w1t[...] + znd_ref[...] @ w1b[...] + b1_ref[...], 0.0)
    neg_ref[...] = hn @ w2_ref[...] + b2


def _p5(zs, zd, znd, W1, b1, W2, b2):
    return pl.pallas_call(
        _p5_body,
        out_shape=(jax.ShapeDtypeStruct((NBATCH, 1), _f32),
                   jax.ShapeDtypeStruct((NBATCH, 1), _f32)),
    )(zs, zd, znd, W1[:DZ], W1[DZ:], b1.reshape(1, DH), W2,
      b2.reshape(1, 1))


def kernel(src, dst, neg_dst, x, n_id, msg, t, edge_index, id_mapper,
           mem_table, last_update_table, W_msg, b_msg, W_self, W1, b1, W2, b2):
    src_e = edge_index[0]
    dst_e = edge_index[1]
    m, xg, lu = _p0(n_id, mem_table, x, last_update_table)
    zw_st, zself = _p1t(m, xg, lu.reshape(NSUB, 1), W_msg, W_self)
    eb_st = _p1e(msg, W_msg, b_msg)
    agg_st = _p2(zw_st, eb_st, src_e, dst_e, t, W_msg[DZ + DMSG])
    znew = _p3(zself, agg_st)
    zs, zd, znd, ms, md = _p4(src, dst, neg_dst, id_mapper, znew, m)
    pos, neg = _p5(zs, zd, znd, W1, b1, W2, b2)
    return pos, neg, ms, md


# speedup vs baseline: 1.5794x; 1.1103x over previous
import jax
import jax.numpy as jnp
from jax import lax
from jax.experimental import pallas as pl
from jax.experimental.pallas import tpu as pltpu
from jax.experimental.pallas import tpu_sc as plsc

NG = 100000
NSUB = 10000
NE = 160000
DM = 128
DF = 128
DMSG = 16
NBATCH = 2048
DZ = DM + DF
DH = 128

NC, NSC, LANES = 2, 16, 16
NW = NC * NSC
DHALF = DZ // NC

_f32 = jnp.float32
_i32 = jnp.int32


def _sc_mesh():
    return plsc.VectorSubcoreMesh(core_axis_name="c", subcore_axis_name="s",
                                  num_cores=NC, num_subcores=NSC)


_P0_K = 80
_P0_NBLK = NSUB // _P0_K


def _p0_body(nid_hbm, mem_hbm, x_hbm, lut_hbm, m_hbm, xg_hbm, lu_hbm,
             idx_v, mbuf, xbuf, lubuf, lut_tbl, sem):
    c = lax.axis_index("c")
    s = lax.axis_index("s")
    wid = s * NC + c
    pltpu.sync_copy(lut_hbm, lut_tbl)
    nblk = jnp.where(wid < _P0_NBLK - (_P0_NBLK // NW) * NW,
                     _P0_NBLK // NW + 1, _P0_NBLK // NW)

    def blk(i, carry):
        r0 = (wid + NW * i) * _P0_K
        pltpu.sync_copy(nid_hbm.at[pl.ds(r0, _P0_K)], idx_v)
        pltpu.async_copy(mem_hbm.at[idx_v], mbuf, sem).wait()
        pltpu.sync_copy(mbuf, m_hbm.at[pl.ds(r0, _P0_K), :])
        pltpu.async_copy(x_hbm.at[idx_v], xbuf, sem).wait()
        pltpu.sync_copy(xbuf, xg_hbm.at[pl.ds(r0, _P0_K), :])
        for j in range(_P0_K // LANES):
            sl = pl.ds(j * LANES, LANES)
            lubuf[sl] = plsc.load_gather(lut_tbl, [idx_v[sl]])
        pltpu.sync_copy(lubuf, lu_hbm.at[pl.ds(r0, _P0_K)])
        return carry

    lax.fori_loop(0, nblk, blk, 0)


def _p0(n_id, mem_table, x, last_update_table):
    return pl.kernel(
        _p0_body,
        out_type=(jax.ShapeDtypeStruct((NSUB, DM), _f32),
                  jax.ShapeDtypeStruct((NSUB, DF), _f32),
                  jax.ShapeDtypeStruct((NSUB,), _f32)),
        mesh=_sc_mesh(),
        compiler_params=pltpu.CompilerParams(needs_layout_passes=False),
        scratch_types=[
            pltpu.VMEM((_P0_K,), _i32),
            pltpu.VMEM((_P0_K, DM), _f32),
            pltpu.VMEM((_P0_K, DF), _f32),
            pltpu.VMEM((_P0_K,), _f32),
            pltpu.VMEM((NG,), _f32),
            pltpu.SemaphoreType.DMA,
        ],
    )(n_id, mem_table, x, last_update_table)


_P1T_BN = 2000


def _p1t_body(m_ref, xg_ref, lu_ref, a1, a2, wt_ref, s1, s2, zw_ref, zself_ref):
    mm = m_ref[...]
    xx = xg_ref[...]
    zw_ref[...] = mm @ a1[...] + xx @ a2[...] - lu_ref[...] * wt_ref[...]
    zself_ref[...] = mm @ s1[...] + xx @ s2[...]


def _p1t(m, xg, lu2, W_msg, W_self):
    nb = NSUB // _P1T_BN
    return pl.pallas_call(
        _p1t_body,
        grid=(NC, nb),
        in_specs=[
            pl.BlockSpec((_P1T_BN, DM), lambda c, i: (i, 0)),
            pl.BlockSpec((_P1T_BN, DF), lambda c, i: (i, 0)),
            pl.BlockSpec((_P1T_BN, 1), lambda c, i: (i, 0)),
            pl.BlockSpec((DM, DHALF), lambda c, i: (0, c)),
            pl.BlockSpec((DF, DHALF), lambda c, i: (0, c)),
            pl.BlockSpec((1, DHALF), lambda c, i: (0, c)),
            pl.BlockSpec((DM, DHALF), lambda c, i: (0, c)),
            pl.BlockSpec((DF, DHALF), lambda c, i: (0, c)),
        ],
        out_specs=(
            pl.BlockSpec((_P1T_BN, DHALF), lambda c, i: (c * (NSUB // _P1T_BN) + i, 0)),
            pl.BlockSpec((_P1T_BN, DHALF), lambda c, i: (i, c)),
        ),
        out_shape=(jax.ShapeDtypeStruct((NC * NSUB, DHALF), _f32),
                   jax.ShapeDtypeStruct((NSUB, DZ), _f32)),
    )(m, xg, lu2, W_msg[:DM], W_msg[DM:DZ], W_msg[DZ + DMSG].reshape(1, DZ),
      W_self[:DM], W_self[DM:])


_P1E_BE = 2000


def _p1e_body(msg_ref, c_ref, bm_ref, eb_ref):
    eb_ref[...] = msg_ref[...] @ c_ref[...] + bm_ref[...]


def _p1e(msg, W_msg, b_msg):
    ne = NE // _P1E_BE
    return pl.pallas_call(
        _p1e_body,
        grid=(ne,),
        in_specs=[
            pl.BlockSpec((_P1E_BE, DMSG), lambda i: (i, 0)),
            pl.BlockSpec((DMSG, DZ), lambda i: (0, 0)),
            pl.BlockSpec((1, DZ), lambda i: (0, 0)),
        ],
        out_specs=pl.BlockSpec((_P1E_BE, DZ), lambda i: (i, 0)),
        out_shape=jax.ShapeDtypeStruct((NE, DZ), _f32),
    )(msg, W_msg[DZ:DZ + DMSG], b_msg.reshape(1, DZ))


_P2_K = 128
_P2_NBLK = NE // _P2_K
_P2_WAVE = 3
_P2_NWAVES = 26
_P2_EXTRA0 = _P2_NWAVES * _P2_WAVE * NSC
_P2_R0 = 624
_P2_CHUNKS = 5


def _p2_body(zw_hbm, eb_hbm, srce_hbm, dste_hbm, t_hbm, wt_hbm, agg_hbm, *scr):
    W = _P2_WAVE
    sidx = scr[0:W]
    didx = scr[W:2 * W]
    buf = scr[2 * W:3 * W]
    tbuf = scr[3 * W:4 * W]
    wtbuf = scr[4 * W]
    aggsh = scr[4 * W + 1]
    lsem = scr[4 * W + 2:5 * W + 2]
    gsem = scr[5 * W + 2:6 * W + 2]
    c = lax.axis_index("c")
    s = lax.axis_index("s")
    pltpu.sync_copy(wt_hbm.at[pl.ds(c * DHALF, DHALF)], wtbuf)
    wvs = [wtbuf[pl.ds(j * LANES, LANES)] for j in range(DHALF // LANES)]

    def zrow(r, carry):
        for j in range(DHALF // LANES):
            buf[0][r, pl.ds(j * LANES, LANES)] = jnp.zeros((LANES,), _f32)
        return carry

    lax.fori_loop(0, _P2_K, zrow, 0)
    r0 = s * _P2_R0
    for k in range(_P2_CHUNKS):
        pltpu.sync_copy(buf[0], aggsh.at[pl.ds(r0 + k * _P2_K, _P2_K), :])
    plsc.subcore_barrier()

    zoff = c * NSUB

    def fire_loads(k, e0):
        return [
            pltpu.async_copy(srce_hbm.at[pl.ds(e0, _P2_K)], sidx[k], lsem[k]),
            pltpu.async_copy(dste_hbm.at[pl.ds(e0, _P2_K)], didx[k], lsem[k]),
            pltpu.async_copy(
                eb_hbm.at[pl.ds(e0, _P2_K), pl.ds(c * DHALF, DHALF)], buf[k],
                lsem[k]),
            pltpu.async_copy(t_hbm.at[pl.ds(e0, _P2_K)],
                             tbuf[k].at[pl.ds(0, _P2_K)], lsem[k]),
        ]

    def compute_block(k):
        def rrow(r, carry2):
            tv = tbuf[k][pl.ds(r, LANES)][0]
            for j in range(DHALF // LANES):
                sl2 = pl.ds(j * LANES, LANES)
                buf[k][r, sl2] = jnp.maximum(buf[k][r, sl2] + tv * wvs[j], 0.0)
            return carry2

        lax.fori_loop(0, _P2_K, rrow, 0)

    def wait_loads(k):
        pltpu.make_async_copy(srce_hbm.at[pl.ds(0, _P2_K)], sidx[k],
                              lsem[k]).wait()
        pltpu.make_async_copy(dste_hbm.at[pl.ds(0, _P2_K)], didx[k],
                              lsem[k]).wait()
        pltpu.make_async_copy(eb_hbm.at[pl.ds(0, _P2_K), pl.ds(0, DHALF)],
                              buf[k], lsem[k]).wait()
        pltpu.make_async_copy(t_hbm.at[pl.ds(0, _P2_K)],
                              tbuf[k].at[pl.ds(0, _P2_K)], lsem[k]).wait()

    for k in range(W):
        fire_loads(k, (s + NSC * k) * _P2_K)

    def wave(g, carry):
        gds = []
        for k in range(W):
            wait_loads(k)
            for j in range(_P2_K // LANES):
                sl = pl.ds(j * LANES, LANES)
                sidx[k][sl] = sidx[k][sl] + zoff
            gds.append(pltpu.async_copy(zw_hbm.at[sidx[k]], buf[k], gsem[k],
                                        add=True))
        for k in range(W):
            gds[k].wait()
            compute_block(k)
            pltpu.sync_copy(buf[k], aggsh.at[didx[k]], add=True)
            nb = s + NSC * (W * (g + 1) + k)
            e0n = jnp.where(g + 1 < _P2_NWAVES, nb * _P2_K, 0)
            fire_loads(k, e0n)
        return carry

    lax.fori_loop(0, _P2_NWAVES, wave, 0)
    for k in range(W):
        wait_loads(k)

    def tail(i, carry):
        e0 = (_P2_EXTRA0 + s) * _P2_K
        for d in fire_loads(0, e0):
            d.wait()
        for j in range(_P2_K // LANES):
            sl = pl.ds(j * LANES, LANES)
            sidx[0][sl] = sidx[0][sl] + zoff
        pltpu.async_copy(zw_hbm.at[sidx[0]], buf[0], gsem[0], add=True).wait()
        compute_block(0)
        pltpu.sync_copy(buf[0], aggsh.at[didx[0]], add=True)
        return carry

    lax.fori_loop(0, jnp.where(s < _P2_NBLK - _P2_EXTRA0, 1, 0), tail, 0)
    plsc.subcore_barrier()

    for k in range(_P2_CHUNKS):
        pltpu.sync_copy(aggsh.at[pl.ds(r0 + k * _P2_K, _P2_K), :], buf[0])
        pltpu.sync_copy(buf[0],
                        agg_hbm.at[pl.ds(c * NSUB + r0 + k * _P2_K, _P2_K), :])


def _p2(zw_st, eb_st, src_e, dst_e, t, wt):
    W = _P2_WAVE
    scratch = (
        [pltpu.VMEM((_P2_K,), _i32) for _ in range(W)]
        + [pltpu.VMEM((_P2_K,), _i32) for _ in range(W)]
        + [pltpu.VMEM((_P2_K, DHALF), _f32) for _ in range(W)]
        + [pltpu.VMEM((_P2_K + LANES,), _f32) for _ in range(W)]
        + [pltpu.VMEM((DHALF,), _f32)]
        + [pltpu.VMEM_SHARED((NSUB, DHALF), _f32)]
        + [pltpu.SemaphoreType.DMA for _ in range(2 * W)]
    )
    return pl.kernel(
        _p2_body,
        out_type=jax.ShapeDtypeStruct((NC * NSUB, DHALF), _f32),
        mesh=_sc_mesh(),
        compiler_params=pltpu.CompilerParams(needs_layout_passes=False),
        scratch_types=scratch,
    )(zw_st, eb_st, src_e, dst_e, t, wt)


_P3_BN = 2000


def _p3_body(zself_ref, agg_ref, znew_ref):
    znew_ref[...] = jnp.maximum(zself_ref[...] + agg_ref[...], 0.0)


def _p3(zself, agg_st):
    nb = NSUB // _P3_BN
    return pl.pallas_call(
        _p3_body,
        grid=(NC, nb),
        in_specs=[
            pl.BlockSpec((_P3_BN, DHALF), lambda c, i: (i, c)),
            pl.BlockSpec((_P3_BN, DHALF), lambda c, i: (c * (NSUB // _P3_BN) + i, 0)),
        ],
        out_specs=pl.BlockSpec((_P3_BN, DHALF), lambda c, i: (i, c)),
        out_shape=jax.ShapeDtypeStruct((NSUB, DZ), _f32),
    )(zself, agg_st)


_P4_PB = NBATCH // NW
_P4_H = 32


def _p4_body(src_hbm, dst_hbm, nd_hbm, idmap_hbm, znew_hbm, m_hbm,
             zs_hbm, zd_hbm, znd_hbm, ms_hbm, md_hbm,
             idmap_tbl, iidx, gbuf, zrows, mrows, sem):
    c = lax.axis_index("c")
    s = lax.axis_index("s")
    wid = s * NC + c
    pltpu.sync_copy(idmap_hbm, idmap_tbl)

    def do(idx_hbm, zout, mout):
        for h in range(_P4_PB // _P4_H):
            b0 = wid * _P4_PB + h * _P4_H
            pltpu.sync_copy(idx_hbm.at[pl.ds(b0, _P4_H)], iidx)
            for j in range(_P4_H // LANES):
                sl = pl.ds(j * LANES, LANES)
                gbuf[sl] = plsc.load_gather(idmap_tbl, [iidx[sl]])
            pltpu.async_copy(znew_hbm.at[gbuf], zrows, sem).wait()
            pltpu.sync_copy(zrows, zout.at[pl.ds(b0, _P4_H), :])
            if mout is not None:
                pltpu.async_copy(m_hbm.at[gbuf], mrows, sem).wait()
                pltpu.sync_copy(mrows, mout.at[pl.ds(b0, _P4_H), :])

    do(src_hbm, zs_hbm, ms_hbm)
    do(dst_hbm, zd_hbm, md_hbm)
    do(nd_hbm, znd_hbm, None)


def _p4(src, dst, neg_dst, id_mapper, znew, m):
    return pl.kernel(
        _p4_body,
        out_type=(jax.ShapeDtypeStruct((NBATCH, DZ), _f32),
                  jax.ShapeDtypeStruct((NBATCH, DZ), _f32),
                  jax.ShapeDtypeStruct((NBATCH, DZ), _f32),
                  jax.ShapeDtypeStruct((NBATCH, DM), _f32),
                  jax.ShapeDtypeStruct((NBATCH, DM), _f32)),
        mesh=_sc_mesh(),
        compiler_params=pltpu.CompilerParams(needs_layout_passes=False),
        scratch_types=[
            pltpu.VMEM((NG,), _i32),
            pltpu.VMEM((_P4_H,), _i32),
            pltpu.VMEM((_P4_H,), _i32),
            pltpu.VMEM((_P4_H, DZ), _f32),
            pltpu.VMEM((_P4_H, DM), _f32),
            pltpu.SemaphoreType.DMA,
        ],
    )(src, dst, neg_dst, id_mapper, znew, m)


def _p5_body(zs_ref, zd_ref, znd_ref, w1t, w1b, b1_ref, w2_ref, b2_ref,
             pos_ref, neg_ref):
    zs = zs_ref[...]
    b2 = b2_ref[0, 0]
    hp = jnp.maximum(zs @ w1t[...] + zd_ref[...] @ w1b[...] + b1_ref[...], 0.0)
    pos_ref[...] = hp @ w2_ref[...] + b2
    hn = jnp.maximum(zs @ w1t[...] + znd_ref[...] @ w1b[...] + b1_ref[...], 0.0)
    neg_ref[...] = hn @ w2_ref[...] + b2


def _p5(zs, zd, znd, W1, b1, W2, b2):
    return pl.pallas_call(
        _p5_body,
        out_shape=(jax.ShapeDtypeStruct((NBATCH, 1), _f32),
                   jax.ShapeDtypeStruct((NBATCH, 1), _f32)),
    )(zs, zd, znd, W1[:DZ], W1[DZ:], b1.reshape(1, DH), W2,
      b2.reshape(1, 1))


def kernel(src, dst, neg_dst, x, n_id, msg, t, edge_index, id_mapper,
           mem_table, last_update_table, W_msg, b_msg, W_self, W1, b1, W2, b2):
    src_e = edge_index[0]
    dst_e = edge_index[1]
    m, xg, lu = _p0(n_id, mem_table, x, last_update_table)
    zw_st, zself = _p1t(m, xg, lu.reshape(NSUB, 1), W_msg, W_self)
    eb_st = _p1e(msg, W_msg, b_msg)
    agg_st = _p2(zw_st, eb_st, src_e, dst_e, t, W_msg[DZ + DMSG])
    znew = _p3(zself, agg_st)
    zs, zd, znd, ms, md = _p4(src, dst, neg_dst, id_mapper, znew, m)
    pos, neg = _p5(zs, zd, znd, W1, b1, W2, b2)
    return pos, neg, ms, md
